# Initial kernel scaffold; baseline (speedup 1.0000x reference)
import jax
import jax.numpy as jnp
from jax.experimental import pallas as pl


def kernel(x, edge_index, W1, b1, W2, b2, Wc, bc):
    raise NotImplementedError("write your pallas kernel here")


# speedup vs baseline: 16.4741x; 16.4741x over previous
import functools

import jax
import jax.numpy as jnp
from jax import lax
from jax.experimental import pallas as pl
from jax.experimental.pallas import tpu as pltpu
from jax.experimental.pallas import tpu_sc as plsc

NC, NS, LANES = 2, 16, 16
NT = NC * NS
CK = 64
ZR = 64
BLK = 512


def _pad_to(v, m):
    return (v + m - 1) // m * m


def _make_deg_kernel(n_pad, cpt, w):
    stripe = n_pad // NS
    mesh = plsc.VectorSubcoreMesh(core_axis_name="c", subcore_axis_name="s")

    @functools.partial(
        pl.kernel,
        out_type=jax.ShapeDtypeStruct((NC * n_pad, w), jnp.float32),
        mesh=mesh,
        scratch_types=[
            pltpu.VMEM((cpt, CK), jnp.int32),
            pltpu.VMEM((CK, w), jnp.float32),
            pltpu.VMEM((ZR, w), jnp.float32),
            pltpu.VMEM_SHARED((n_pad, w), jnp.float32),
        ],
    )
    def deg_kernel(dst_hbm, out_hbm, dstbuf, ones_v, zbuf, hist):
        c = lax.axis_index("c")
        s = lax.axis_index("s")
        tile = c * NS + s
        one16 = jnp.full((LANES,), 1.0, jnp.float32)
        zero16 = jnp.zeros((LANES,), jnp.float32)
        for r in range(CK):
            for q in range(w // LANES):
                ones_v[r, pl.ds(q * LANES, LANES)] = one16
        for r in range(ZR):
            for q in range(w // LANES):
                zbuf[r, pl.ds(q * LANES, LANES)] = zero16
        for t in range(stripe // ZR):
            pltpu.sync_copy(zbuf, hist.at[pl.ds(s * stripe + t * ZR, ZR)])
        pltpu.sync_copy(dst_hbm.at[pl.ds(tile * cpt, cpt)], dstbuf)
        plsc.subcore_barrier()
        for j in range(cpt):
            pltpu.sync_copy(ones_v, hist.at[dstbuf.at[j]], add=True)
        plsc.subcore_barrier()
        pltpu.sync_copy(
            hist.at[pl.ds(s * stripe, stripe)],
            out_hbm.at[pl.ds(c * n_pad + s * stripe, stripe)],
        )

    return deg_kernel


def _make_scat_kernel(n_pad, hid, cpt):
    stripe = n_pad // NS
    mesh = plsc.VectorSubcoreMesh(core_axis_name="c", subcore_axis_name="s")

    @functools.partial(
        pl.kernel,
        out_type=jax.ShapeDtypeStruct((NC * n_pad, hid), jnp.float32),
        mesh=mesh,
        scratch_types=[
            pltpu.VMEM((cpt, CK), jnp.int32),
            pltpu.VMEM((cpt, CK), jnp.int32),
            pltpu.VMEM((CK, hid), jnp.float32),
            pltpu.VMEM((CK, hid), jnp.float32),
            pltpu.VMEM_SHARED((n_pad, hid), jnp.float32),
            pltpu.SemaphoreType.DMA,
            pltpu.SemaphoreType.DMA,
        ],
    )
    def scat_kernel(g_hbm, src_hbm, dst_hbm, out_hbm,
                    srcbuf, dstbuf, rows0, rows1, acc, gsem0, gsem1):
        c = lax.axis_index("c")
        s = lax.axis_index("s")
        tile = c * NS + s
        zero16 = jnp.zeros((LANES,), jnp.float32)
        for r in range(CK):
            for q in range(hid // LANES):
                rows0[r, pl.ds(q * LANES, LANES)] = zero16
        for t in range(stripe // CK):
            pltpu.sync_copy(rows0, acc.at[pl.ds(s * stripe + t * CK, CK)])
        pltpu.sync_copy(src_hbm.at[pl.ds(tile * cpt, cpt)], srcbuf)
        pltpu.sync_copy(dst_hbm.at[pl.ds(tile * cpt, cpt)], dstbuf)
        plsc.subcore_barrier()
        rows = (rows0, rows1)
        gsems = (gsem0, gsem1)
        gd = [None, None]
        for j in range(cpt):
            gd[j % 2] = pltpu.async_copy(g_hbm.at[srcbuf.at[j]], rows[j % 2],
                                         gsems[j % 2])
            if j >= 1:
                gd[(j - 1) % 2].wait()
                pltpu.sync_copy(rows[(j - 1) % 2], acc.at[dstbuf.at[j - 1]],
                                add=True)
        last = cpt - 1
        gd[last % 2].wait()
        pltpu.sync_copy(rows[last % 2], acc.at[dstbuf.at[last]], add=True)
        plsc.subcore_barrier()
        pltpu.sync_copy(
            acc.at[pl.ds(s * stripe, stripe)],
            out_hbm.at[pl.ds(c * n_pad + s * stripe, stripe)],
        )

    return scat_kernel


def _dinv_block(dega_ref, degb_ref):
    deg = dega_ref[:, 0:1] + degb_ref[:, 0:1] + 1.0
    return lax.rsqrt(jnp.maximum(deg, 1.0))


def _tc1_call(x, w1, degp, n, n_pad):
    d_in, hid = w1.shape
    nb = n_pad // BLK

    def body(x_ref, w_ref, dega_ref, degb_ref, o_ref):
        i = pl.program_id(0)
        dinv = _dinv_block(dega_ref, degb_ref)
        m = jnp.dot(x_ref[...], w_ref[...], preferred_element_type=jnp.float32,
                    precision=lax.Precision.HIGHEST)
        row = lax.broadcasted_iota(jnp.int32, (BLK, 1), 0) + i * BLK
        o_ref[...] = jnp.where(row < n, m * dinv, 0.0)

    return pl.pallas_call(
        body,
        grid=(nb,),
        in_specs=[
            pl.BlockSpec((BLK, d_in), lambda i: (i, 0)),
            pl.BlockSpec((d_in, hid), lambda i: (0, 0)),
            pl.BlockSpec((BLK, hid), lambda i: (i, 0)),
            pl.BlockSpec((BLK, hid), lambda i: (i + nb, 0)),
        ],
        out_specs=pl.BlockSpec((BLK, hid), lambda i: (i, 0)),
        out_shape=jax.ShapeDtypeStruct((n_pad, hid), jnp.float32),
    )(x, w1, degp, degp)


def _tc2_call(g1, parts, degp, b1, w2, n, n_pad):
    hid, hid2 = w2.shape
    nb = n_pad // BLK

    def body(g_ref, p0_ref, p1_ref, dega_ref, degb_ref, b_ref, w_ref, o_ref):
        i = pl.program_id(0)
        dinv = _dinv_block(dega_ref, degb_ref)
        h = jnp.maximum(dinv * (g_ref[...] + p0_ref[...] + p1_ref[...])
                        + b_ref[...], 0.0)
        m = jnp.dot(h, w_ref[...], preferred_element_type=jnp.float32,
                    precision=lax.Precision.HIGHEST)
        row = lax.broadcasted_iota(jnp.int32, (BLK, 1), 0) + i * BLK
        o_ref[...] = jnp.where(row < n, m * dinv, 0.0)

    return pl.pallas_call(
        body,
        grid=(nb,),
        in_specs=[
            pl.BlockSpec((BLK, hid), lambda i: (i, 0)),
            pl.BlockSpec((BLK, hid), lambda i: (i, 0)),
            pl.BlockSpec((BLK, hid), lambda i: (i + nb, 0)),
            pl.BlockSpec((BLK, hid), lambda i: (i, 0)),
            pl.BlockSpec((BLK, hid), lambda i: (i + nb, 0)),
            pl.BlockSpec((1, hid), lambda i: (0, 0)),
            pl.BlockSpec((hid, hid2), lambda i: (0, 0)),
        ],
        out_specs=pl.BlockSpec((BLK, hid2), lambda i: (i, 0)),
        out_shape=jax.ShapeDtypeStruct((n_pad, hid2), jnp.float32),
    )(g1, parts, parts, degp, degp, b1.reshape(1, hid), w2)


def _tc3_call(g2, parts, degp, b2, wc, bc, n, n_pad):
    hid, ncls = wc.shape
    nb = n_pad // BLK

    def body(g_ref, p0_ref, p1_ref, dega_ref, degb_ref, b_ref, w_ref, bc_ref,
             o_ref):
        dinv = _dinv_block(dega_ref, degb_ref)
        h = jnp.maximum(dinv * (g_ref[...] + p0_ref[...] + p1_ref[...])
                        + b_ref[...], 0.0)
        o_ref[...] = jnp.dot(h, w_ref[...], preferred_element_type=jnp.float32,
                             precision=lax.Precision.HIGHEST) + bc_ref[...]

    return pl.pallas_call(
        body,
        grid=(nb,),
        in_specs=[
            pl.BlockSpec((BLK, hid), lambda i: (i, 0)),
            pl.BlockSpec((BLK, hid), lambda i: (i, 0)),
            pl.BlockSpec((BLK, hid), lambda i: (i + nb, 0)),
            pl.BlockSpec((BLK, hid), lambda i: (i, 0)),
            pl.BlockSpec((BLK, hid), lambda i: (i + nb, 0)),
            pl.BlockSpec((1, hid), lambda i: (0, 0)),
            pl.BlockSpec((hid, ncls), lambda i: (0, 0)),
            pl.BlockSpec((1, ncls), lambda i: (0, 0)),
        ],
        out_specs=pl.BlockSpec((BLK, ncls), lambda i: (i, 0)),
        out_shape=jax.ShapeDtypeStruct((n, ncls), jnp.float32),
    )(g2, parts, parts, degp, degp, b2.reshape(1, hid), wc, bc.reshape(1, ncls))


def kernel(x, edge_index, W1, b1, W2, b2, Wc, bc):
    n, _ = x.shape
    hid = W1.shape[1]
    e = edge_index.shape[1]

    n_pad = _pad_to(n + 1, NS * ZR)
    e_pad = _pad_to(e, NT * CK * 8)
    cpt = e_pad // (NT * CK)

    pad_idx = n + jnp.arange(e_pad - e, dtype=jnp.int32) % (n_pad - n)
    src3 = jnp.concatenate([edge_index[0], pad_idx]).reshape(e_pad // CK, CK)
    dst3 = jnp.concatenate([edge_index[1], pad_idx]).reshape(e_pad // CK, CK)

    degp = _make_deg_kernel(n_pad, cpt, hid)(dst3)
    g1 = _tc1_call(x, W1, degp, n, n_pad)
    scat = _make_scat_kernel(n_pad, hid, cpt)
    parts1 = scat(g1, src3, dst3)
    g2 = _tc2_call(g1, parts1, degp, b1, W2, n, n_pad)
    parts2 = scat(g2, src3, dst3)
    return _tc3_call(g2, parts2, degp, b2, Wc, bc, n, n_pad)


# speedup vs baseline: 16.4984x; 1.0015x over previous
import functools

import jax
import jax.numpy as jnp
from jax import lax
from jax.experimental import pallas as pl
from jax.experimental.pallas import tpu as pltpu
from jax.experimental.pallas import tpu_sc as plsc

NC, NS, LANES = 2, 16, 16
NT = NC * NS
CK = 64
ZR = 64
BLK = 512


def _pad_to(v, m):
    return (v + m - 1) // m * m


def _make_deg_kernel(n_pad, cpt, w):
    stripe = n_pad // NS
    mesh = plsc.VectorSubcoreMesh(core_axis_name="c", subcore_axis_name="s")

    @functools.partial(
        pl.kernel,
        out_type=jax.ShapeDtypeStruct((NC * n_pad, w), jnp.float32),
        mesh=mesh,
        scratch_types=[
            pltpu.VMEM((cpt, CK), jnp.int32),
            pltpu.VMEM((CK, w), jnp.float32),
            pltpu.VMEM((ZR, w), jnp.float32),
            pltpu.VMEM_SHARED((n_pad, w), jnp.float32),
            pltpu.SemaphoreType.DMA,
        ],
    )
    def deg_kernel(dst_hbm, out_hbm, dstbuf, ones_v, zbuf, hist, sem):
        c = lax.axis_index("c")
        s = lax.axis_index("s")
        tile = c * NS + s
        one16 = jnp.full((LANES,), 1.0, jnp.float32)
        zero16 = jnp.zeros((LANES,), jnp.float32)
        for r in range(CK):
            for q in range(w // LANES):
                ones_v[r, pl.ds(q * LANES, LANES)] = one16
        for r in range(ZR):
            for q in range(w // LANES):
                zbuf[r, pl.ds(q * LANES, LANES)] = zero16
        for t in range(stripe // ZR):
            pltpu.sync_copy(zbuf, hist.at[pl.ds(s * stripe + t * ZR, ZR)])
        pltpu.sync_copy(dst_hbm.at[pl.ds(tile * cpt, cpt)], dstbuf)
        plsc.subcore_barrier()
        descs = [pltpu.async_copy(ones_v, hist.at[dstbuf.at[j]], sem, add=True)
                 for j in range(cpt)]
        for d in descs:
            d.wait()
        plsc.subcore_barrier()
        pltpu.sync_copy(
            hist.at[pl.ds(s * stripe, stripe)],
            out_hbm.at[pl.ds(c * n_pad + s * stripe, stripe)],
        )

    return deg_kernel


def _make_scat_kernel(n_pad, hid, cpt):
    stripe = n_pad // NS
    mesh = plsc.VectorSubcoreMesh(core_axis_name="c", subcore_axis_name="s")

    @functools.partial(
        pl.kernel,
        out_type=jax.ShapeDtypeStruct((NC * n_pad, hid), jnp.float32),
        mesh=mesh,
        scratch_types=[
            pltpu.VMEM((cpt, CK), jnp.int32),
            pltpu.VMEM((cpt, CK), jnp.int32),
            pltpu.VMEM((CK, hid), jnp.float32),
            pltpu.VMEM((CK, hid), jnp.float32),
            pltpu.VMEM_SHARED((n_pad, hid), jnp.float32),
            pltpu.SemaphoreType.DMA,
            pltpu.SemaphoreType.DMA,
            pltpu.SemaphoreType.DMA,
            pltpu.SemaphoreType.DMA,
        ],
    )
    def scat_kernel(g_hbm, src_hbm, dst_hbm, out_hbm,
                    srcbuf, dstbuf, rows0, rows1, acc,
                    gsem0, gsem1, ssem0, ssem1):
        c = lax.axis_index("c")
        s = lax.axis_index("s")
        tile = c * NS + s
        zero16 = jnp.zeros((LANES,), jnp.float32)
        for r in range(CK):
            for q in range(hid // LANES):
                rows0[r, pl.ds(q * LANES, LANES)] = zero16
        for t in range(stripe // CK):
            pltpu.sync_copy(rows0, acc.at[pl.ds(s * stripe + t * CK, CK)])
        pltpu.sync_copy(src_hbm.at[pl.ds(tile * cpt, cpt)], srcbuf)
        pltpu.sync_copy(dst_hbm.at[pl.ds(tile * cpt, cpt)], dstbuf)
        plsc.subcore_barrier()
        rows = (rows0, rows1)
        gsems = (gsem0, gsem1)
        ssems = (ssem0, ssem1)
        gd = [None, None]
        sd = [None, None]
        for j in range(cpt):
            if j >= 2:
                sd[j % 2].wait()
            gd[j % 2] = pltpu.async_copy(g_hbm.at[srcbuf.at[j]], rows[j % 2],
                                         gsems[j % 2])
            if j >= 1:
                gd[(j - 1) % 2].wait()
                sd[(j - 1) % 2] = pltpu.async_copy(
                    rows[(j - 1) % 2], acc.at[dstbuf.at[j - 1]],
                    ssems[(j - 1) % 2], add=True)
        last = cpt - 1
        gd[last % 2].wait()
        sd[last % 2] = pltpu.async_copy(rows[last % 2], acc.at[dstbuf.at[last]],
                                        ssems[last % 2], add=True)
        sd[(last - 1) % 2].wait()
        sd[last % 2].wait()
        plsc.subcore_barrier()
        pltpu.sync_copy(
            acc.at[pl.ds(s * stripe, stripe)],
            out_hbm.at[pl.ds(c * n_pad + s * stripe, stripe)],
        )

    return scat_kernel


def _dinv_block(dega_ref, degb_ref):
    deg = dega_ref[:, 0:1] + degb_ref[:, 0:1] + 1.0
    return lax.rsqrt(jnp.maximum(deg, 1.0))


def _tc1_call(x, w1, degp, n, n_pad):
    d_in, hid = w1.shape
    nb = n_pad // BLK

    def body(x_ref, w_ref, dega_ref, degb_ref, o_ref):
        i = pl.program_id(0)
        dinv = _dinv_block(dega_ref, degb_ref)
        m = jnp.dot(x_ref[...], w_ref[...], preferred_element_type=jnp.float32,
                    precision=lax.Precision.HIGHEST)
        row = lax.broadcasted_iota(jnp.int32, (BLK, 1), 0) + i * BLK
        o_ref[...] = jnp.where(row < n, m * dinv, 0.0)

    return pl.pallas_call(
        body,
        grid=(nb,),
        in_specs=[
            pl.BlockSpec((BLK, d_in), lambda i: (i, 0)),
            pl.BlockSpec((d_in, hid), lambda i: (0, 0)),
            pl.BlockSpec((BLK, hid), lambda i: (i, 0)),
            pl.BlockSpec((BLK, hid), lambda i: (i + nb, 0)),
        ],
        out_specs=pl.BlockSpec((BLK, hid), lambda i: (i, 0)),
        out_shape=jax.ShapeDtypeStruct((n_pad, hid), jnp.float32),
    )(x, w1, degp, degp)


def _tc2_call(g1, parts, degp, b1, w2, n, n_pad):
    hid, hid2 = w2.shape
    nb = n_pad // BLK

    def body(g_ref, p0_ref, p1_ref, dega_ref, degb_ref, b_ref, w_ref, o_ref):
        i = pl.program_id(0)
        dinv = _dinv_block(dega_ref, degb_ref)
        h = jnp.maximum(dinv * (g_ref[...] + p0_ref[...] + p1_ref[...])
                        + b_ref[...], 0.0)
        m = jnp.dot(h, w_ref[...], preferred_element_type=jnp.float32,
                    precision=lax.Precision.HIGHEST)
        row = lax.broadcasted_iota(jnp.int32, (BLK, 1), 0) + i * BLK
        o_ref[...] = jnp.where(row < n, m * dinv, 0.0)

    return pl.pallas_call(
        body,
        grid=(nb,),
        in_specs=[
            pl.BlockSpec((BLK, hid), lambda i: (i, 0)),
            pl.BlockSpec((BLK, hid), lambda i: (i, 0)),
            pl.BlockSpec((BLK, hid), lambda i: (i + nb, 0)),
            pl.BlockSpec((BLK, hid), lambda i: (i, 0)),
            pl.BlockSpec((BLK, hid), lambda i: (i + nb, 0)),
            pl.BlockSpec((1, hid), lambda i: (0, 0)),
            pl.BlockSpec((hid, hid2), lambda i: (0, 0)),
        ],
        out_specs=pl.BlockSpec((BLK, hid2), lambda i: (i, 0)),
        out_shape=jax.ShapeDtypeStruct((n_pad, hid2), jnp.float32),
    )(g1, parts, parts, degp, degp, b1.reshape(1, hid), w2)


def _tc3_call(g2, parts, degp, b2, wc, bc, n, n_pad):
    hid, ncls = wc.shape
    nb = n_pad // BLK

    def body(g_ref, p0_ref, p1_ref, dega_ref, degb_ref, b_ref, w_ref, bc_ref,
             o_ref):
        dinv = _dinv_block(dega_ref, degb_ref)
        h = jnp.maximum(dinv * (g_ref[...] + p0_ref[...] + p1_ref[...])
                        + b_ref[...], 0.0)
        o_ref[...] = jnp.dot(h, w_ref[...], preferred_element_type=jnp.float32,
                             precision=lax.Precision.HIGHEST) + bc_ref[...]

    return pl.pallas_call(
        body,
        grid=(nb,),
        in_specs=[
            pl.BlockSpec((BLK, hid), lambda i: (i, 0)),
            pl.BlockSpec((BLK, hid), lambda i: (i, 0)),
            pl.BlockSpec((BLK, hid), lambda i: (i + nb, 0)),
            pl.BlockSpec((BLK, hid), lambda i: (i, 0)),
            pl.BlockSpec((BLK, hid), lambda i: (i + nb, 0)),
            pl.BlockSpec((1, hid), lambda i: (0, 0)),
            pl.BlockSpec((hid, ncls), lambda i: (0, 0)),
            pl.BlockSpec((1, ncls), lambda i: (0, 0)),
        ],
        out_specs=pl.BlockSpec((BLK, ncls), lambda i: (i, 0)),
        out_shape=jax.ShapeDtypeStruct((n, ncls), jnp.float32),
    )(g2, parts, parts, degp, degp, b2.reshape(1, hid), wc, bc.reshape(1, ncls))


def kernel(x, edge_index, W1, b1, W2, b2, Wc, bc):
    n, _ = x.shape
    hid = W1.shape[1]
    e = edge_index.shape[1]

    n_pad = _pad_to(n + 1, NS * ZR)
    e_pad = _pad_to(e, NT * CK * 8)
    cpt = e_pad // (NT * CK)

    pad_idx = n + jnp.arange(e_pad - e, dtype=jnp.int32) % (n_pad - n)
    src3 = jnp.concatenate([edge_index[0], pad_idx]).reshape(e_pad // CK, CK)
    dst3 = jnp.concatenate([edge_index[1], pad_idx]).reshape(e_pad // CK, CK)

    degp = _make_deg_kernel(n_pad, cpt, hid)(dst3)
    g1 = _tc1_call(x, W1, degp, n, n_pad)
    scat = _make_scat_kernel(n_pad, hid, cpt)
    parts1 = scat(g1, src3, dst3)
    g2 = _tc2_call(g1, parts1, degp, b1, W2, n, n_pad)
    parts2 = scat(g2, src3, dst3)
    return _tc3_call(g2, parts2, degp, b2, Wc, bc, n, n_pad)


# speedup vs baseline: 18.0061x; 1.0914x over previous
import functools

import jax
import jax.numpy as jnp
from jax import lax
from jax.experimental import pallas as pl
from jax.experimental.pallas import tpu as pltpu
from jax.experimental.pallas import tpu_sc as plsc

NC, NS, LANES = 2, 16, 16
NT = NC * NS
CK = 128
ZR = 64
BLK = 512


def _pad_to(v, m):
    return (v + m - 1) // m * m


def _make_deg_kernel(n_pad, cpt, w):
    stripe = n_pad // NS
    mesh = plsc.VectorSubcoreMesh(core_axis_name="c", subcore_axis_name="s")

    @functools.partial(
        pl.kernel,
        out_type=jax.ShapeDtypeStruct((NC * n_pad, w), jnp.float32),
        mesh=mesh,
        scratch_types=[
            pltpu.VMEM((cpt, CK), jnp.int32),
            pltpu.VMEM((CK, w), jnp.float32),
            pltpu.VMEM((ZR, w), jnp.float32),
            pltpu.VMEM_SHARED((n_pad, w), jnp.float32),
            pltpu.SemaphoreType.DMA,
        ],
    )
    def deg_kernel(dst_hbm, out_hbm, dstbuf, ones_v, zbuf, hist, sem):
        c = lax.axis_index("c")
        s = lax.axis_index("s")
        tile = c * NS + s
        one16 = jnp.full((LANES,), 1.0, jnp.float32)
        zero16 = jnp.zeros((LANES,), jnp.float32)
        for r in range(CK):
            for q in range(w // LANES):
                ones_v[r, pl.ds(q * LANES, LANES)] = one16
        for r in range(ZR):
            for q in range(w // LANES):
                zbuf[r, pl.ds(q * LANES, LANES)] = zero16
        for t in range(stripe // ZR):
            pltpu.sync_copy(zbuf, hist.at[pl.ds(s * stripe + t * ZR, ZR)])
        pltpu.sync_copy(dst_hbm.at[pl.ds(tile * cpt, cpt)], dstbuf)
        plsc.subcore_barrier()
        descs = [pltpu.async_copy(ones_v, hist.at[dstbuf.at[j]], sem, add=True)
                 for j in range(cpt)]
        for d in descs:
            d.wait()
        plsc.subcore_barrier()
        pltpu.sync_copy(
            hist.at[pl.ds(s * stripe, stripe)],
            out_hbm.at[pl.ds(c * n_pad + s * stripe, stripe)],
        )

    return deg_kernel


def _make_scat_kernel(n_pad, hid, cpt):
    stripe = n_pad // NS
    mesh = plsc.VectorSubcoreMesh(core_axis_name="c", subcore_axis_name="s")

    @functools.partial(
        pl.kernel,
        out_type=jax.ShapeDtypeStruct((NC * n_pad, hid), jnp.float32),
        mesh=mesh,
        scratch_types=[
            pltpu.VMEM((cpt, CK), jnp.int32),
            pltpu.VMEM((cpt, CK), jnp.int32),
            pltpu.VMEM((CK, hid), jnp.float32),
            pltpu.VMEM((CK, hid), jnp.float32),
            pltpu.VMEM_SHARED((n_pad, hid), jnp.float32),
            pltpu.SemaphoreType.DMA,
            pltpu.SemaphoreType.DMA,
            pltpu.SemaphoreType.DMA,
            pltpu.SemaphoreType.DMA,
        ],
    )
    def scat_kernel(g_hbm, src_hbm, dst_hbm, out_hbm,
                    srcbuf, dstbuf, rows0, rows1, acc,
                    gsem0, gsem1, ssem0, ssem1):
        c = lax.axis_index("c")
        s = lax.axis_index("s")
        tile = c * NS + s
        zero16 = jnp.zeros((LANES,), jnp.float32)
        for r in range(CK):
            for q in range(hid // LANES):
                rows0[r, pl.ds(q * LANES, LANES)] = zero16
        for t in range(stripe // CK):
            pltpu.sync_copy(rows0, acc.at[pl.ds(s * stripe + t * CK, CK)])
        pltpu.sync_copy(src_hbm.at[pl.ds(tile * cpt, cpt)], srcbuf)
        pltpu.sync_copy(dst_hbm.at[pl.ds(tile * cpt, cpt)], dstbuf)
        plsc.subcore_barrier()
        rows = (rows0, rows1)
        gsems = (gsem0, gsem1)
        ssems = (ssem0, ssem1)
        gd = [None, None]
        sd = [None, None]
        for j in range(cpt):
            if j >= 2:
                sd[j % 2].wait()
            gd[j % 2] = pltpu.async_copy(g_hbm.at[srcbuf.at[j]], rows[j % 2],
                                         gsems[j % 2])
            if j >= 1:
                gd[(j - 1) % 2].wait()
                sd[(j - 1) % 2] = pltpu.async_copy(
                    rows[(j - 1) % 2], acc.at[dstbuf.at[j - 1]],
                    ssems[(j - 1) % 2], add=True)
        last = cpt - 1
        gd[last % 2].wait()
        sd[last % 2] = pltpu.async_copy(rows[last % 2], acc.at[dstbuf.at[last]],
                                        ssems[last % 2], add=True)
        sd[(last - 1) % 2].wait()
        sd[last % 2].wait()
        plsc.subcore_barrier()
        pltpu.sync_copy(
            acc.at[pl.ds(s * stripe, stripe)],
            out_hbm.at[pl.ds(c * n_pad + s * stripe, stripe)],
        )

    return scat_kernel


def _dinv_block(dega_ref, degb_ref):
    deg = dega_ref[:, 0:1] + degb_ref[:, 0:1] + 1.0
    return lax.rsqrt(jnp.maximum(deg, 1.0))


def _tc1_call(x, w1, degp, n, n_pad):
    d_in, hid = w1.shape
    nb = n_pad // BLK

    def body(x_ref, w_ref, dega_ref, degb_ref, o_ref):
        i = pl.program_id(0)
        dinv = _dinv_block(dega_ref, degb_ref)
        m = jnp.dot(x_ref[...], w_ref[...], preferred_element_type=jnp.float32,
                    precision=lax.Precision.HIGHEST)
        row = lax.broadcasted_iota(jnp.int32, (BLK, 1), 0) + i * BLK
        o_ref[...] = jnp.where(row < n, m * dinv, 0.0)

    return pl.pallas_call(
        body,
        grid=(nb,),
        in_specs=[
            pl.BlockSpec((BLK, d_in), lambda i: (i, 0)),
            pl.BlockSpec((d_in, hid), lambda i: (0, 0)),
            pl.BlockSpec((BLK, hid), lambda i: (i, 0)),
            pl.BlockSpec((BLK, hid), lambda i: (i + nb, 0)),
        ],
        out_specs=pl.BlockSpec((BLK, hid), lambda i: (i, 0)),
        out_shape=jax.ShapeDtypeStruct((n_pad, hid), jnp.float32),
    )(x, w1, degp, degp)


def _tc2_call(g1, parts, degp, b1, w2, n, n_pad):
    hid, hid2 = w2.shape
    nb = n_pad // BLK

    def body(g_ref, p0_ref, p1_ref, dega_ref, degb_ref, b_ref, w_ref, o_ref):
        i = pl.program_id(0)
        dinv = _dinv_block(dega_ref, degb_ref)
        h = jnp.maximum(dinv * (g_ref[...] + p0_ref[...] + p1_ref[...])
                        + b_ref[...], 0.0)
        m = jnp.dot(h, w_ref[...], preferred_element_type=jnp.float32,
                    precision=lax.Precision.HIGHEST)
        row = lax.broadcasted_iota(jnp.int32, (BLK, 1), 0) + i * BLK
        o_ref[...] = jnp.where(row < n, m * dinv, 0.0)

    return pl.pallas_call(
        body,
        grid=(nb,),
        in_specs=[
            pl.BlockSpec((BLK, hid), lambda i: (i, 0)),
            pl.BlockSpec((BLK, hid), lambda i: (i, 0)),
            pl.BlockSpec((BLK, hid), lambda i: (i + nb, 0)),
            pl.BlockSpec((BLK, hid), lambda i: (i, 0)),
            pl.BlockSpec((BLK, hid), lambda i: (i + nb, 0)),
            pl.BlockSpec((1, hid), lambda i: (0, 0)),
            pl.BlockSpec((hid, hid2), lambda i: (0, 0)),
        ],
        out_specs=pl.BlockSpec((BLK, hid2), lambda i: (i, 0)),
        out_shape=jax.ShapeDtypeStruct((n_pad, hid2), jnp.float32),
    )(g1, parts, parts, degp, degp, b1.reshape(1, hid), w2)


def _tc3_call(g2, parts, degp, b2, wc, bc, n, n_pad):
    hid, ncls = wc.shape
    nb = n_pad // BLK

    def body(g_ref, p0_ref, p1_ref, dega_ref, degb_ref, b_ref, w_ref, bc_ref,
             o_ref):
        dinv = _dinv_block(dega_ref, degb_ref)
        h = jnp.maximum(dinv * (g_ref[...] + p0_ref[...] + p1_ref[...])
                        + b_ref[...], 0.0)
        o_ref[...] = jnp.dot(h, w_ref[...], preferred_element_type=jnp.float32,
                             precision=lax.Precision.HIGHEST) + bc_ref[...]

    return pl.pallas_call(
        body,
        grid=(nb,),
        in_specs=[
            pl.BlockSpec((BLK, hid), lambda i: (i, 0)),
            pl.BlockSpec((BLK, hid), lambda i: (i, 0)),
            pl.BlockSpec((BLK, hid), lambda i: (i + nb, 0)),
            pl.BlockSpec((BLK, hid), lambda i: (i, 0)),
            pl.BlockSpec((BLK, hid), lambda i: (i + nb, 0)),
            pl.BlockSpec((1, hid), lambda i: (0, 0)),
            pl.BlockSpec((hid, ncls), lambda i: (0, 0)),
            pl.BlockSpec((1, ncls), lambda i: (0, 0)),
        ],
        out_specs=pl.BlockSpec((BLK, ncls), lambda i: (i, 0)),
        out_shape=jax.ShapeDtypeStruct((n, ncls), jnp.float32),
    )(g2, parts, parts, degp, degp, b2.reshape(1, hid), wc, bc.reshape(1, ncls))


def kernel(x, edge_index, W1, b1, W2, b2, Wc, bc):
    n, _ = x.shape
    hid = W1.shape[1]
    e = edge_index.shape[1]

    n_pad = _pad_to(n + 1, NS * ZR)
    e_pad = _pad_to(e, NT * CK * 8)
    cpt = e_pad // (NT * CK)

    pad_idx = n + jnp.arange(e_pad - e, dtype=jnp.int32) % (n_pad - n)
    src3 = jnp.concatenate([edge_index[0], pad_idx]).reshape(e_pad // CK, CK)
    dst3 = jnp.concatenate([edge_index[1], pad_idx]).reshape(e_pad // CK, CK)

    degp = _make_deg_kernel(n_pad, cpt, hid)(dst3)
    g1 = _tc1_call(x, W1, degp, n, n_pad)
    scat = _make_scat_kernel(n_pad, hid, cpt)
    parts1 = scat(g1, src3, dst3)
    g2 = _tc2_call(g1, parts1, degp, b1, W2, n, n_pad)
    parts2 = scat(g2, src3, dst3)
    return _tc3_call(g2, parts2, degp, b2, Wc, bc, n, n_pad)


# speedup vs baseline: 18.8240x; 1.0454x over previous
import functools

import jax
import jax.numpy as jnp
from jax import lax
from jax.experimental import pallas as pl
from jax.experimental.pallas import tpu as pltpu
from jax.experimental.pallas import tpu_sc as plsc

NC, NS, LANES = 2, 16, 16
NT = NC * NS
CK = 128
ZR = 64
BLK = 512


def _pad_to(v, m):
    return (v + m - 1) // m * m


def _make_deg_kernel(n_pad, cpt, w):
    stripe = n_pad // NS
    mesh = plsc.VectorSubcoreMesh(core_axis_name="c", subcore_axis_name="s")

    @functools.partial(
        pl.kernel,
        out_type=jax.ShapeDtypeStruct((NC * n_pad, w), jnp.float32),
        mesh=mesh,
        scratch_types=[
            pltpu.VMEM((cpt, CK), jnp.int32),
            pltpu.VMEM((CK, w), jnp.float32),
            pltpu.VMEM((ZR, w), jnp.float32),
            pltpu.VMEM_SHARED((n_pad, w), jnp.float32),
            pltpu.SemaphoreType.DMA,
        ],
    )
    def deg_kernel(dst_hbm, out_hbm, dstbuf, ones_v, zbuf, hist, sem):
        c = lax.axis_index("c")
        s = lax.axis_index("s")
        tile = c * NS + s
        one16 = jnp.full((LANES,), 1.0, jnp.float32)
        zero16 = jnp.zeros((LANES,), jnp.float32)
        for r in range(CK):
            for q in range(w // LANES):
                ones_v[r, pl.ds(q * LANES, LANES)] = one16
        for r in range(ZR):
            for q in range(w // LANES):
                zbuf[r, pl.ds(q * LANES, LANES)] = zero16
        for t in range(stripe // ZR):
            pltpu.sync_copy(zbuf, hist.at[pl.ds(s * stripe + t * ZR, ZR)])
        pltpu.sync_copy(dst_hbm.at[pl.ds(tile * cpt, cpt)], dstbuf)
        plsc.subcore_barrier()
        descs = [pltpu.async_copy(ones_v, hist.at[dstbuf.at[j]], sem, add=True)
                 for j in range(cpt)]
        for d in descs:
            d.wait()
        plsc.subcore_barrier()
        pltpu.sync_copy(
            hist.at[pl.ds(s * stripe, stripe)],
            out_hbm.at[pl.ds(c * n_pad + s * stripe, stripe)],
        )

    return deg_kernel


def _make_scat_kernel(n_pad, hid, cpt):
    stripe = n_pad // NS
    mesh = plsc.VectorSubcoreMesh(core_axis_name="c", subcore_axis_name="s")

    @functools.partial(
        pl.kernel,
        out_type=jax.ShapeDtypeStruct((NC * n_pad, hid), jnp.float32),
        mesh=mesh,
        scratch_types=[
            pltpu.VMEM((cpt, CK), jnp.int32),
            pltpu.VMEM((cpt, CK), jnp.int32),
            pltpu.VMEM((CK, hid), jnp.float32),
            pltpu.VMEM((CK, hid), jnp.float32),
            pltpu.VMEM_SHARED((n_pad, hid), jnp.float32),
            pltpu.SemaphoreType.DMA,
            pltpu.SemaphoreType.DMA,
            pltpu.SemaphoreType.DMA,
            pltpu.SemaphoreType.DMA,
        ],
    )
    def scat_kernel(g_hbm, src_hbm, dst_hbm, out_hbm,
                    srcbuf, dstbuf, rows0, rows1, acc,
                    gsem0, gsem1, ssem0, ssem1):
        c = lax.axis_index("c")
        s = lax.axis_index("s")
        tile = c * NS + s
        zero16 = jnp.zeros((LANES,), jnp.float32)
        for r in range(CK):
            for q in range(hid // LANES):
                rows0[r, pl.ds(q * LANES, LANES)] = zero16
        for t in range(stripe // CK):
            pltpu.sync_copy(rows0, acc.at[pl.ds(s * stripe + t * CK, CK)])
        pltpu.sync_copy(src_hbm.at[pl.ds(tile * cpt, cpt)], srcbuf)
        pltpu.sync_copy(dst_hbm.at[pl.ds(tile * cpt, cpt)], dstbuf)
        plsc.subcore_barrier()
        rows = (rows0, rows1)
        gsems = (gsem0, gsem1)
        ssems = (ssem0, ssem1)
        gd = [None, None]
        sd = [None, None]
        for j in range(cpt):
            if j >= 2:
                sd[j % 2].wait()
            gd[j % 2] = pltpu.async_copy(g_hbm.at[srcbuf.at[j]], rows[j % 2],
                                         gsems[j % 2])
            if j >= 1:
                gd[(j - 1) % 2].wait()
                sd[(j - 1) % 2] = pltpu.async_copy(
                    rows[(j - 1) % 2], acc.at[dstbuf.at[j - 1]],
                    ssems[(j - 1) % 2], add=True)
        last = cpt - 1
        gd[last % 2].wait()
        sd[last % 2] = pltpu.async_copy(rows[last % 2], acc.at[dstbuf.at[last]],
                                        ssems[last % 2], add=True)
        sd[(last - 1) % 2].wait()
        sd[last % 2].wait()
        plsc.subcore_barrier()
        pltpu.sync_copy(
            acc.at[pl.ds(s * stripe, stripe)],
            out_hbm.at[pl.ds(c * n_pad + s * stripe, stripe)],
        )

    return scat_kernel


def _dinv_block(dega_ref, degb_ref):
    deg = dega_ref[:, 0:1] + degb_ref[:, 0:1] + 1.0
    return lax.rsqrt(jnp.maximum(deg, 1.0))


def _tc0_call(x, w1, n, n_pad):
    d_in, hid = w1.shape
    nb = n_pad // BLK

    def body(x_ref, w_ref, o_ref):
        i = pl.program_id(0)
        m = jnp.dot(x_ref[...], w_ref[...], preferred_element_type=jnp.float32,
                    precision=lax.Precision.HIGHEST)
        row = lax.broadcasted_iota(jnp.int32, (BLK, 1), 0) + i * BLK
        o_ref[...] = jnp.where(row < n, m, 0.0)

    return pl.pallas_call(
        body,
        grid=(nb,),
        in_specs=[
            pl.BlockSpec((BLK, d_in), lambda i: (i, 0)),
            pl.BlockSpec((d_in, hid), lambda i: (0, 0)),
        ],
        out_specs=pl.BlockSpec((BLK, hid), lambda i: (i, 0)),
        out_shape=jax.ShapeDtypeStruct((n_pad, hid), jnp.float32),
    )(x, w1)


def _scale_call(m1, degp, n_pad, hid):
    nb = n_pad // BLK

    def body(m_ref, dega_ref, degb_ref, g_ref, dinv_ref):
        dinv = _dinv_block(dega_ref, degb_ref)
        g_ref[...] = m_ref[...] * dinv
        dinv_ref[...] = dinv

    return pl.pallas_call(
        body,
        grid=(nb,),
        in_specs=[
            pl.BlockSpec((BLK, hid), lambda i: (i, 0)),
            pl.BlockSpec((BLK, hid), lambda i: (i, 0)),
            pl.BlockSpec((BLK, hid), lambda i: (i + nb, 0)),
        ],
        out_specs=[
            pl.BlockSpec((BLK, hid), lambda i: (i, 0)),
            pl.BlockSpec((BLK, 1), lambda i: (i, 0)),
        ],
        out_shape=[
            jax.ShapeDtypeStruct((n_pad, hid), jnp.float32),
            jax.ShapeDtypeStruct((n_pad, 1), jnp.float32),
        ],
    )(m1, degp, degp)


def _tc2_call(g1, parts, dinv_arr, b1, w2, n, n_pad):
    hid, hid2 = w2.shape
    nb = n_pad // BLK

    def body(g_ref, p0_ref, p1_ref, dinv_ref, b_ref, w_ref, o_ref):
        i = pl.program_id(0)
        dinv = dinv_ref[...]
        h = jnp.maximum(dinv * (g_ref[...] + p0_ref[...] + p1_ref[...])
                        + b_ref[...], 0.0)
        m = jnp.dot(h, w_ref[...], preferred_element_type=jnp.float32,
                    precision=lax.Precision.HIGHEST)
        row = lax.broadcasted_iota(jnp.int32, (BLK, 1), 0) + i * BLK
        o_ref[...] = jnp.where(row < n, m * dinv, 0.0)

    return pl.pallas_call(
        body,
        grid=(nb,),
        in_specs=[
            pl.BlockSpec((BLK, hid), lambda i: (i, 0)),
            pl.BlockSpec((BLK, hid), lambda i: (i, 0)),
            pl.BlockSpec((BLK, hid), lambda i: (i + nb, 0)),
            pl.BlockSpec((BLK, 1), lambda i: (i, 0)),
            pl.BlockSpec((1, hid), lambda i: (0, 0)),
            pl.BlockSpec((hid, hid2), lambda i: (0, 0)),
        ],
        out_specs=pl.BlockSpec((BLK, hid2), lambda i: (i, 0)),
        out_shape=jax.ShapeDtypeStruct((n_pad, hid2), jnp.float32),
    )(g1, parts, parts, dinv_arr, b1.reshape(1, hid), w2)


def _tc3_call(g2, parts, dinv_arr, b2, wc, bc, n, n_pad):
    hid, ncls = wc.shape
    nb = n_pad // BLK

    def body(g_ref, p0_ref, p1_ref, dinv_ref, b_ref, w_ref, bc_ref, o_ref):
        dinv = dinv_ref[...]
        h = jnp.maximum(dinv * (g_ref[...] + p0_ref[...] + p1_ref[...])
                        + b_ref[...], 0.0)
        o_ref[...] = jnp.dot(h, w_ref[...], preferred_element_type=jnp.float32,
                             precision=lax.Precision.HIGHEST) + bc_ref[...]

    return pl.pallas_call(
        body,
        grid=(nb,),
        in_specs=[
            pl.BlockSpec((BLK, hid), lambda i: (i, 0)),
            pl.BlockSpec((BLK, hid), lambda i: (i, 0)),
            pl.BlockSpec((BLK, hid), lambda i: (i + nb, 0)),
            pl.BlockSpec((BLK, 1), lambda i: (i, 0)),
            pl.BlockSpec((1, hid), lambda i: (0, 0)),
            pl.BlockSpec((hid, ncls), lambda i: (0, 0)),
            pl.BlockSpec((1, ncls), lambda i: (0, 0)),
        ],
        out_specs=pl.BlockSpec((BLK, ncls), lambda i: (i, 0)),
        out_shape=jax.ShapeDtypeStruct((n, ncls), jnp.float32),
    )(g2, parts, parts, dinv_arr, b2.reshape(1, hid), wc, bc.reshape(1, ncls))


def kernel(x, edge_index, W1, b1, W2, b2, Wc, bc):
    n, _ = x.shape
    hid = W1.shape[1]
    e = edge_index.shape[1]

    n_pad = _pad_to(n + 1, NS * ZR)
    e_pad = _pad_to(e, NT * CK * 8)
    cpt = e_pad // (NT * CK)

    pad_idx = n + jnp.arange(e_pad - e, dtype=jnp.int32) % (n_pad - n)
    src3 = jnp.concatenate([edge_index[0], pad_idx]).reshape(e_pad // CK, CK)
    dst3 = jnp.concatenate([edge_index[1], pad_idx]).reshape(e_pad // CK, CK)

    degp = _make_deg_kernel(n_pad, cpt, hid)(dst3)
    m1 = _tc0_call(x, W1, n, n_pad)
    g1, dinv_arr = _scale_call(m1, degp, n_pad, hid)
    scat = _make_scat_kernel(n_pad, hid, cpt)
    parts1 = scat(g1, src3, dst3)
    g2 = _tc2_call(g1, parts1, dinv_arr, b1, W2, n, n_pad)
    parts2 = scat(g2, src3, dst3)
    return _tc3_call(g2, parts2, dinv_arr, b2, Wc, bc, n, n_pad)


# speedup vs baseline: 20.3906x; 1.0832x over previous
import functools

import jax
import jax.numpy as jnp
from jax import lax
from jax.experimental import pallas as pl
from jax.experimental.pallas import tpu as pltpu
from jax.experimental.pallas import tpu_sc as plsc

NC, NS, LANES = 2, 16, 16
NT = NC * NS
CK = 128
ZR = 64
BLK = 1024


def _pad_to(v, m):
    return (v + m - 1) // m * m


def _make_deg_kernel(n_pad, cpt, w):
    stripe = n_pad // NS
    mesh = plsc.VectorSubcoreMesh(core_axis_name="c", subcore_axis_name="s")

    @functools.partial(
        pl.kernel,
        out_type=jax.ShapeDtypeStruct((NC * n_pad, w), jnp.float32),
        mesh=mesh,
        scratch_types=[
            pltpu.VMEM((cpt, CK), jnp.int32),
            pltpu.VMEM((CK, w), jnp.float32),
            pltpu.VMEM((ZR, w), jnp.float32),
            pltpu.VMEM_SHARED((n_pad, w), jnp.float32),
            pltpu.SemaphoreType.DMA,
        ],
    )
    def deg_kernel(dst_hbm, out_hbm, dstbuf, ones_v, zbuf, hist, sem):
        c = lax.axis_index("c")
        s = lax.axis_index("s")
        tile = c * NS + s
        one16 = jnp.full((LANES,), 1.0, jnp.float32)
        zero16 = jnp.zeros((LANES,), jnp.float32)
        for r in range(CK):
            for q in range(w // LANES):
                ones_v[r, pl.ds(q * LANES, LANES)] = one16
        for r in range(ZR):
            for q in range(w // LANES):
                zbuf[r, pl.ds(q * LANES, LANES)] = zero16
        for t in range(stripe // ZR):
            pltpu.sync_copy(zbuf, hist.at[pl.ds(s * stripe + t * ZR, ZR)])
        pltpu.sync_copy(dst_hbm.at[pl.ds(tile * cpt, cpt)], dstbuf)
        plsc.subcore_barrier()
        descs = [pltpu.async_copy(ones_v, hist.at[dstbuf.at[j]], sem, add=True)
                 for j in range(cpt)]
        for d in descs:
            d.wait()
        plsc.subcore_barrier()
        pltpu.sync_copy(
            hist.at[pl.ds(s * stripe, stripe)],
            out_hbm.at[pl.ds(c * n_pad + s * stripe, stripe)],
        )

    return deg_kernel


def _make_scat_kernel(n_pad, hid, cpt):
    stripe = n_pad // NS
    mesh = plsc.VectorSubcoreMesh(core_axis_name="c", subcore_axis_name="s")

    @functools.partial(
        pl.kernel,
        out_type=jax.ShapeDtypeStruct((NC * n_pad, hid), jnp.float32),
        mesh=mesh,
        scratch_types=[
            pltpu.VMEM((cpt, CK), jnp.int32),
            pltpu.VMEM((cpt, CK), jnp.int32),
            pltpu.VMEM((CK, hid), jnp.float32),
            pltpu.VMEM((CK, hid), jnp.float32),
            pltpu.VMEM_SHARED((n_pad, hid), jnp.float32),
            pltpu.SemaphoreType.DMA,
            pltpu.SemaphoreType.DMA,
            pltpu.SemaphoreType.DMA,
            pltpu.SemaphoreType.DMA,
        ],
    )
    def scat_kernel(g_hbm, src_hbm, dst_hbm, out_hbm,
                    srcbuf, dstbuf, rows0, rows1, acc,
                    gsem0, gsem1, ssem0, ssem1):
        c = lax.axis_index("c")
        s = lax.axis_index("s")
        tile = c * NS + s
        zero16 = jnp.zeros((LANES,), jnp.float32)
        for r in range(CK):
            for q in range(hid // LANES):
                rows0[r, pl.ds(q * LANES, LANES)] = zero16
        for t in range(stripe // CK):
            pltpu.sync_copy(rows0, acc.at[pl.ds(s * stripe + t * CK, CK)])
        pltpu.sync_copy(src_hbm.at[pl.ds(tile * cpt, cpt)], srcbuf)
        pltpu.sync_copy(dst_hbm.at[pl.ds(tile * cpt, cpt)], dstbuf)
        plsc.subcore_barrier()
        rows = (rows0, rows1)
        gsems = (gsem0, gsem1)
        ssems = (ssem0, ssem1)
        gd = [None, None]
        sd = [None, None]
        for j in range(cpt):
            if j >= 2:
                sd[j % 2].wait()
            gd[j % 2] = pltpu.async_copy(g_hbm.at[srcbuf.at[j]], rows[j % 2],
                                         gsems[j % 2])
            if j >= 1:
                gd[(j - 1) % 2].wait()
                sd[(j - 1) % 2] = pltpu.async_copy(
                    rows[(j - 1) % 2], acc.at[dstbuf.at[j - 1]],
                    ssems[(j - 1) % 2], add=True)
        last = cpt - 1
        gd[last % 2].wait()
        sd[last % 2] = pltpu.async_copy(rows[last % 2], acc.at[dstbuf.at[last]],
                                        ssems[last % 2], add=True)
        sd[(last - 1) % 2].wait()
        sd[last % 2].wait()
        plsc.subcore_barrier()
        pltpu.sync_copy(
            acc.at[pl.ds(s * stripe, stripe)],
            out_hbm.at[pl.ds(c * n_pad + s * stripe, stripe)],
        )

    return scat_kernel


def _dinv_block(dega_ref, degb_ref):
    deg = dega_ref[:, 0:1] + degb_ref[:, 0:1] + 1.0
    return lax.rsqrt(jnp.maximum(deg, 1.0))


def _tc0_call(x, w1, n, n_pad):
    d_in, hid = w1.shape
    nb = n_pad // BLK

    def body(x_ref, w_ref, o_ref):
        i = pl.program_id(0)
        m = jnp.dot(x_ref[...], w_ref[...], preferred_element_type=jnp.float32)
        row = lax.broadcasted_iota(jnp.int32, (BLK, 1), 0) + i * BLK
        o_ref[...] = jnp.where(row < n, m, 0.0)

    return pl.pallas_call(
        body,
        grid=(nb,),
        in_specs=[
            pl.BlockSpec((BLK, d_in), lambda i: (i, 0)),
            pl.BlockSpec((d_in, hid), lambda i: (0, 0)),
        ],
        out_specs=pl.BlockSpec((BLK, hid), lambda i: (i, 0)),
        out_shape=jax.ShapeDtypeStruct((n_pad, hid), jnp.float32),
    )(x, w1)


def _scale_call(m1, degp, n_pad, hid):
    nb = n_pad // BLK

    def body(m_ref, dega_ref, degb_ref, g_ref, dinv_ref):
        dinv = _dinv_block(dega_ref, degb_ref)
        g_ref[...] = m_ref[...] * dinv
        dinv_ref[...] = dinv

    return pl.pallas_call(
        body,
        grid=(nb,),
        in_specs=[
            pl.BlockSpec((BLK, hid), lambda i: (i, 0)),
            pl.BlockSpec((BLK, hid), lambda i: (i, 0)),
            pl.BlockSpec((BLK, hid), lambda i: (i + nb, 0)),
        ],
        out_specs=[
            pl.BlockSpec((BLK, hid), lambda i: (i, 0)),
            pl.BlockSpec((BLK, 1), lambda i: (i, 0)),
        ],
        out_shape=[
            jax.ShapeDtypeStruct((n_pad, hid), jnp.float32),
            jax.ShapeDtypeStruct((n_pad, 1), jnp.float32),
        ],
    )(m1, degp, degp)


def _tc2_call(g1, parts, dinv_arr, b1, w2, n, n_pad):
    hid, hid2 = w2.shape
    nb = n_pad // BLK

    def body(g_ref, p0_ref, p1_ref, dinv_ref, b_ref, w_ref, o_ref):
        i = pl.program_id(0)
        dinv = dinv_ref[...]
        h = jnp.maximum(dinv * (g_ref[...] + p0_ref[...] + p1_ref[...])
                        + b_ref[...], 0.0)
        m = jnp.dot(h, w_ref[...], preferred_element_type=jnp.float32)
        row = lax.broadcasted_iota(jnp.int32, (BLK, 1), 0) + i * BLK
        o_ref[...] = jnp.where(row < n, m * dinv, 0.0)

    return pl.pallas_call(
        body,
        grid=(nb,),
        in_specs=[
            pl.BlockSpec((BLK, hid), lambda i: (i, 0)),
            pl.BlockSpec((BLK, hid), lambda i: (i, 0)),
            pl.BlockSpec((BLK, hid), lambda i: (i + nb, 0)),
            pl.BlockSpec((BLK, 1), lambda i: (i, 0)),
            pl.BlockSpec((1, hid), lambda i: (0, 0)),
            pl.BlockSpec((hid, hid2), lambda i: (0, 0)),
        ],
        out_specs=pl.BlockSpec((BLK, hid2), lambda i: (i, 0)),
        out_shape=jax.ShapeDtypeStruct((n_pad, hid2), jnp.float32),
    )(g1, parts, parts, dinv_arr, b1.reshape(1, hid), w2)


def _tc3_call(g2, parts, dinv_arr, b2, wc, bc, n, n_pad):
    hid, ncls = wc.shape
    nb = n_pad // BLK

    def body(g_ref, p0_ref, p1_ref, dinv_ref, b_ref, w_ref, bc_ref, o_ref):
        dinv = dinv_ref[...]
        h = jnp.maximum(dinv * (g_ref[...] + p0_ref[...] + p1_ref[...])
                        + b_ref[...], 0.0)
        o_ref[...] = jnp.dot(h, w_ref[...], preferred_element_type=jnp.float32) + bc_ref[...]

    return pl.pallas_call(
        body,
        grid=(nb,),
        in_specs=[
            pl.BlockSpec((BLK, hid), lambda i: (i, 0)),
            pl.BlockSpec((BLK, hid), lambda i: (i, 0)),
            pl.BlockSpec((BLK, hid), lambda i: (i + nb, 0)),
            pl.BlockSpec((BLK, 1), lambda i: (i, 0)),
            pl.BlockSpec((1, hid), lambda i: (0, 0)),
            pl.BlockSpec((hid, ncls), lambda i: (0, 0)),
            pl.BlockSpec((1, ncls), lambda i: (0, 0)),
        ],
        out_specs=pl.BlockSpec((BLK, ncls), lambda i: (i, 0)),
        out_shape=jax.ShapeDtypeStruct((n, ncls), jnp.float32),
    )(g2, parts, parts, dinv_arr, b2.reshape(1, hid), wc, bc.reshape(1, ncls))


def kernel(x, edge_index, W1, b1, W2, b2, Wc, bc):
    n, _ = x.shape
    hid = W1.shape[1]
    e = edge_index.shape[1]

    n_pad = _pad_to(n + 1, NS * ZR)
    e_pad = _pad_to(e, NT * CK * 8)
    cpt = e_pad // (NT * CK)

    pad_idx = n + jnp.arange(e_pad - e, dtype=jnp.int32) % (n_pad - n)
    src3 = jnp.concatenate([edge_index[0], pad_idx]).reshape(e_pad // CK, CK)
    dst3 = jnp.concatenate([edge_index[1], pad_idx]).reshape(e_pad // CK, CK)

    degp = _make_deg_kernel(n_pad, cpt, hid)(dst3)
    m1 = _tc0_call(x, W1, n, n_pad)
    g1, dinv_arr = _scale_call(m1, degp, n_pad, hid)
    scat = _make_scat_kernel(n_pad, hid, cpt)
    parts1 = scat(g1, src3, dst3)
    g2 = _tc2_call(g1, parts1, dinv_arr, b1, W2, n, n_pad)
    parts2 = scat(g2, src3, dst3)
    return _tc3_call(g2, parts2, dinv_arr, b2, Wc, bc, n, n_pad)


# speedup vs baseline: 21.1928x; 1.0393x over previous
import functools

import jax
import jax.numpy as jnp
from jax import lax
from jax.experimental import pallas as pl
from jax.experimental.pallas import tpu as pltpu
from jax.experimental.pallas import tpu_sc as plsc

NC, NS, LANES = 2, 16, 16
NT = NC * NS
CK = 128
ZR = 64
BLK = 1024


def _pad_to(v, m):
    return (v + m - 1) // m * m


def _make_deg_kernel(n_pad, cpt, w):
    stripe = n_pad // NS
    mesh = plsc.VectorSubcoreMesh(core_axis_name="c", subcore_axis_name="s")

    @functools.partial(
        pl.kernel,
        out_type=jax.ShapeDtypeStruct((NC * n_pad, w), jnp.float32),
        mesh=mesh,
        scratch_types=[
            pltpu.VMEM((cpt, CK), jnp.int32),
            pltpu.VMEM((CK, w), jnp.float32),
            pltpu.VMEM((ZR, w), jnp.float32),
            pltpu.VMEM_SHARED((n_pad, w), jnp.float32),
            pltpu.SemaphoreType.DMA,
        ],
    )
    def deg_kernel(dst_hbm, out_hbm, dstbuf, ones_v, zbuf, hist, sem):
        c = lax.axis_index("c")
        s = lax.axis_index("s")
        tile = c * NS + s
        one16 = jnp.full((LANES,), 1.0, jnp.float32)
        zero16 = jnp.zeros((LANES,), jnp.float32)
        for r in range(CK):
            for q in range(w // LANES):
                ones_v[r, pl.ds(q * LANES, LANES)] = one16
        for r in range(ZR):
            for q in range(w // LANES):
                zbuf[r, pl.ds(q * LANES, LANES)] = zero16
        zds = [pltpu.async_copy(zbuf, hist.at[pl.ds(s * stripe + t * ZR, ZR)],
                                sem)
               for t in range(stripe // ZR)]
        pltpu.sync_copy(dst_hbm.at[pl.ds(tile * cpt, cpt)], dstbuf)
        for d in zds:
            d.wait()
        plsc.subcore_barrier()
        descs = [pltpu.async_copy(ones_v, hist.at[dstbuf.at[j]], sem, add=True)
                 for j in range(cpt)]
        for d in descs:
            d.wait()
        plsc.subcore_barrier()
        pltpu.sync_copy(
            hist.at[pl.ds(s * stripe, stripe)],
            out_hbm.at[pl.ds(c * n_pad + s * stripe, stripe)],
        )

    return deg_kernel


def _make_scat_kernel(n_pad, hid, cpt):
    stripe = n_pad // NS
    mesh = plsc.VectorSubcoreMesh(core_axis_name="c", subcore_axis_name="s")

    @functools.partial(
        pl.kernel,
        out_type=jax.ShapeDtypeStruct((NC * n_pad, hid), jnp.float32),
        mesh=mesh,
        scratch_types=[
            pltpu.VMEM((cpt, CK), jnp.int32),
            pltpu.VMEM((cpt, CK), jnp.int32),
            pltpu.VMEM((CK, hid), jnp.float32),
            pltpu.VMEM((CK, hid), jnp.float32),
            pltpu.VMEM_SHARED((n_pad, hid), jnp.float32),
            pltpu.SemaphoreType.DMA,
            pltpu.SemaphoreType.DMA,
            pltpu.SemaphoreType.DMA,
            pltpu.SemaphoreType.DMA,
        ],
    )
    def scat_kernel(g_hbm, src_hbm, dst_hbm, out_hbm,
                    srcbuf, dstbuf, rows0, rows1, acc,
                    gsem0, gsem1, ssem0, ssem1):
        c = lax.axis_index("c")
        s = lax.axis_index("s")
        tile = c * NS + s
        zero16 = jnp.zeros((LANES,), jnp.float32)
        for r in range(CK):
            for q in range(hid // LANES):
                rows1[r, pl.ds(q * LANES, LANES)] = zero16
        zds = [pltpu.async_copy(rows1, acc.at[pl.ds(s * stripe + t * CK, CK)],
                                ssem0)
               for t in range(stripe // CK)]
        pltpu.sync_copy(src_hbm.at[pl.ds(tile * cpt, cpt)], srcbuf)
        pltpu.sync_copy(dst_hbm.at[pl.ds(tile * cpt, cpt)], dstbuf)
        rows = (rows0, rows1)
        gsems = (gsem0, gsem1)
        ssems = (ssem0, ssem1)
        gd = [None, None]
        sd = [None, None]
        gd[0] = pltpu.async_copy(g_hbm.at[srcbuf.at[0]], rows[0], gsems[0])
        for d in zds:
            d.wait()
        plsc.subcore_barrier()
        for j in range(1, cpt):
            if j >= 2:
                sd[j % 2].wait()
            gd[j % 2] = pltpu.async_copy(g_hbm.at[srcbuf.at[j]], rows[j % 2],
                                         gsems[j % 2])
            gd[(j - 1) % 2].wait()
            sd[(j - 1) % 2] = pltpu.async_copy(
                rows[(j - 1) % 2], acc.at[dstbuf.at[j - 1]],
                ssems[(j - 1) % 2], add=True)
        last = cpt - 1
        gd[last % 2].wait()
        sd[last % 2] = pltpu.async_copy(rows[last % 2], acc.at[dstbuf.at[last]],
                                        ssems[last % 2], add=True)
        sd[(last - 1) % 2].wait()
        sd[last % 2].wait()
        plsc.subcore_barrier()
        pltpu.sync_copy(
            acc.at[pl.ds(s * stripe, stripe)],
            out_hbm.at[pl.ds(c * n_pad + s * stripe, stripe)],
        )

    return scat_kernel


def _dinv_block(dega_ref, degb_ref):
    deg = dega_ref[:, 0:1] + degb_ref[:, 0:1] + 1.0
    return lax.rsqrt(jnp.maximum(deg, 1.0))


def _tc0_call(x, w1, n, n_pad):
    d_in, hid = w1.shape
    nb = n_pad // BLK

    def body(x_ref, w_ref, o_ref):
        i = pl.program_id(0)
        m = jnp.dot(x_ref[...], w_ref[...], preferred_element_type=jnp.float32)
        row = lax.broadcasted_iota(jnp.int32, (BLK, 1), 0) + i * BLK
        o_ref[...] = jnp.where(row < n, m, 0.0)

    return pl.pallas_call(
        body,
        grid=(nb,),
        in_specs=[
            pl.BlockSpec((BLK, d_in), lambda i: (i, 0)),
            pl.BlockSpec((d_in, hid), lambda i: (0, 0)),
        ],
        out_specs=pl.BlockSpec((BLK, hid), lambda i: (i, 0)),
        out_shape=jax.ShapeDtypeStruct((n_pad, hid), jnp.float32),
    )(x, w1)


def _scale_call(m1, degp, n_pad, hid):
    nb = n_pad // BLK

    def body(m_ref, dega_ref, degb_ref, g_ref, dinv_ref):
        dinv = _dinv_block(dega_ref, degb_ref)
        g_ref[...] = m_ref[...] * dinv
        dinv_ref[...] = dinv

    return pl.pallas_call(
        body,
        grid=(nb,),
        in_specs=[
            pl.BlockSpec((BLK, hid), lambda i: (i, 0)),
            pl.BlockSpec((BLK, hid), lambda i: (i, 0)),
            pl.BlockSpec((BLK, hid), lambda i: (i + nb, 0)),
        ],
        out_specs=[
            pl.BlockSpec((BLK, hid), lambda i: (i, 0)),
            pl.BlockSpec((BLK, 1), lambda i: (i, 0)),
        ],
        out_shape=[
            jax.ShapeDtypeStruct((n_pad, hid), jnp.float32),
            jax.ShapeDtypeStruct((n_pad, 1), jnp.float32),
        ],
    )(m1, degp, degp)


def _tc2_call(g1, parts, dinv_arr, b1, w2, n, n_pad):
    hid, hid2 = w2.shape
    nb = n_pad // BLK

    def body(g_ref, p0_ref, p1_ref, dinv_ref, b_ref, w_ref, o_ref):
        i = pl.program_id(0)
        dinv = dinv_ref[...]
        h = jnp.maximum(dinv * (g_ref[...] + p0_ref[...] + p1_ref[...])
                        + b_ref[...], 0.0)
        m = jnp.dot(h, w_ref[...], preferred_element_type=jnp.float32)
        row = lax.broadcasted_iota(jnp.int32, (BLK, 1), 0) + i * BLK
        o_ref[...] = jnp.where(row < n, m * dinv, 0.0)

    return pl.pallas_call(
        body,
        grid=(nb,),
        in_specs=[
            pl.BlockSpec((BLK, hid), lambda i: (i, 0)),
            pl.BlockSpec((BLK, hid), lambda i: (i, 0)),
            pl.BlockSpec((BLK, hid), lambda i: (i + nb, 0)),
            pl.BlockSpec((BLK, 1), lambda i: (i, 0)),
            pl.BlockSpec((1, hid), lambda i: (0, 0)),
            pl.BlockSpec((hid, hid2), lambda i: (0, 0)),
        ],
        out_specs=pl.BlockSpec((BLK, hid2), lambda i: (i, 0)),
        out_shape=jax.ShapeDtypeStruct((n_pad, hid2), jnp.float32),
    )(g1, parts, parts, dinv_arr, b1.reshape(1, hid), w2)


def _tc3_call(g2, parts, dinv_arr, b2, wc, bc, n, n_pad):
    hid, ncls = wc.shape
    nb = n_pad // BLK

    def body(g_ref, p0_ref, p1_ref, dinv_ref, b_ref, w_ref, bc_ref, o_ref):
        dinv = dinv_ref[...]
        h = jnp.maximum(dinv * (g_ref[...] + p0_ref[...] + p1_ref[...])
                        + b_ref[...], 0.0)
        o_ref[...] = jnp.dot(h, w_ref[...], preferred_element_type=jnp.float32) + bc_ref[...]

    return pl.pallas_call(
        body,
        grid=(nb,),
        in_specs=[
            pl.BlockSpec((BLK, hid), lambda i: (i, 0)),
            pl.BlockSpec((BLK, hid), lambda i: (i, 0)),
            pl.BlockSpec((BLK, hid), lambda i: (i + nb, 0)),
            pl.BlockSpec((BLK, 1), lambda i: (i, 0)),
            pl.BlockSpec((1, hid), lambda i: (0, 0)),
            pl.BlockSpec((hid, ncls), lambda i: (0, 0)),
            pl.BlockSpec((1, ncls), lambda i: (0, 0)),
        ],
        out_specs=pl.BlockSpec((BLK, ncls), lambda i: (i, 0)),
        out_shape=jax.ShapeDtypeStruct((n, ncls), jnp.float32),
    )(g2, parts, parts, dinv_arr, b2.reshape(1, hid), wc, bc.reshape(1, ncls))


def kernel(x, edge_index, W1, b1, W2, b2, Wc, bc):
    n, _ = x.shape
    hid = W1.shape[1]
    e = edge_index.shape[1]

    n_pad = _pad_to(n + 1, NS * ZR)
    e_pad = _pad_to(e, NT * CK * 8)
    cpt = e_pad // (NT * CK)

    pad_idx = n + jnp.arange(e_pad - e, dtype=jnp.int32) % (n_pad - n)
    src3 = jnp.concatenate([edge_index[0], pad_idx]).reshape(e_pad // CK, CK)
    dst3 = jnp.concatenate([edge_index[1], pad_idx]).reshape(e_pad // CK, CK)

    degp = _make_deg_kernel(n_pad, cpt, hid)(dst3)
    m1 = _tc0_call(x, W1, n, n_pad)
    g1, dinv_arr = _scale_call(m1, degp, n_pad, hid)
    scat = _make_scat_kernel(n_pad, hid, cpt)
    parts1 = scat(g1, src3, dst3)
    g2 = _tc2_call(g1, parts1, dinv_arr, b1, W2, n, n_pad)
    parts2 = scat(g2, src3, dst3)
    return _tc3_call(g2, parts2, dinv_arr, b2, Wc, bc, n, n_pad)


# speedup vs baseline: 21.9711x; 1.0367x over previous
import functools

import jax
import jax.numpy as jnp
from jax import lax
from jax.experimental import pallas as pl
from jax.experimental.pallas import tpu as pltpu
from jax.experimental.pallas import tpu_sc as plsc

NC, NS, LANES = 2, 16, 16
NT = NC * NS
CK = 128
ZR = 64
BLK = 1024


def _pad_to(v, m):
    return (v + m - 1) // m * m


def _make_deg_kernel(n_pad, cpt, ck, w):
    stripe = n_pad // NS
    mesh = plsc.VectorSubcoreMesh(core_axis_name="c", subcore_axis_name="s")

    @functools.partial(
        pl.kernel,
        out_type=jax.ShapeDtypeStruct((NC * n_pad, w), jnp.float32),
        mesh=mesh,
        scratch_types=[
            pltpu.VMEM((cpt, ck), jnp.int32),
            pltpu.VMEM((ck, w), jnp.float32),
            pltpu.VMEM((ZR, w), jnp.float32),
            pltpu.VMEM_SHARED((n_pad, w), jnp.float32),
            pltpu.SemaphoreType.DMA,
        ],
    )
    def deg_kernel(ei_hbm, out_hbm, dstbuf, ones_v, zbuf, hist, sem):
        c = lax.axis_index("c")
        s = lax.axis_index("s")
        tile = c * NS + s
        one16 = jnp.full((LANES,), 1.0, jnp.float32)
        zero16 = jnp.zeros((LANES,), jnp.float32)
        for r in range(ck):
            for q in range(w // LANES):
                ones_v[r, pl.ds(q * LANES, LANES)] = one16
        for r in range(ZR):
            for q in range(w // LANES):
                zbuf[r, pl.ds(q * LANES, LANES)] = zero16
        zds = [pltpu.async_copy(zbuf, hist.at[pl.ds(s * stripe + t * ZR, ZR)],
                                sem)
               for t in range(stripe // ZR)]
        pltpu.sync_copy(ei_hbm.at[1, pl.ds(tile * cpt, cpt)], dstbuf)
        for d in zds:
            d.wait()
        plsc.subcore_barrier()
        descs = [pltpu.async_copy(ones_v, hist.at[dstbuf.at[j]], sem, add=True)
                 for j in range(cpt)]
        for d in descs:
            d.wait()
        plsc.subcore_barrier()
        pltpu.sync_copy(
            hist.at[pl.ds(s * stripe, stripe)],
            out_hbm.at[pl.ds(c * n_pad + s * stripe, stripe)],
        )

    return deg_kernel


def _make_scat_kernel(n_pad, hid, cpt, ck):
    stripe = n_pad // NS
    mesh = plsc.VectorSubcoreMesh(core_axis_name="c", subcore_axis_name="s")

    @functools.partial(
        pl.kernel,
        out_type=jax.ShapeDtypeStruct((NC * n_pad, hid), jnp.float32),
        mesh=mesh,
        scratch_types=[
            pltpu.VMEM((cpt, ck), jnp.int32),
            pltpu.VMEM((cpt, ck), jnp.int32),
            pltpu.VMEM((ck, hid), jnp.float32),
            pltpu.VMEM((ck, hid), jnp.float32),
            pltpu.VMEM_SHARED((n_pad, hid), jnp.float32),
            pltpu.SemaphoreType.DMA,
            pltpu.SemaphoreType.DMA,
            pltpu.SemaphoreType.DMA,
            pltpu.SemaphoreType.DMA,
        ],
    )
    def scat_kernel(g_hbm, ei_hbm, out_hbm,
                    srcbuf, dstbuf, rows0, rows1, acc,
                    gsem0, gsem1, ssem0, ssem1):
        c = lax.axis_index("c")
        s = lax.axis_index("s")
        tile = c * NS + s
        zero16 = jnp.zeros((LANES,), jnp.float32)
        for r in range(ZR):
            for q in range(hid // LANES):
                rows1[r, pl.ds(q * LANES, LANES)] = zero16
        zds = [pltpu.async_copy(rows1.at[pl.ds(0, ZR)],
                                acc.at[pl.ds(s * stripe + t * ZR, ZR)],
                                ssem0)
               for t in range(stripe // ZR)]
        pltpu.sync_copy(ei_hbm.at[0, pl.ds(tile * cpt, cpt)], srcbuf)
        pltpu.sync_copy(ei_hbm.at[1, pl.ds(tile * cpt, cpt)], dstbuf)
        rows = (rows0, rows1)
        gsems = (gsem0, gsem1)
        ssems = (ssem0, ssem1)
        gd = [None, None]
        sd = [None, None]
        gd[0] = pltpu.async_copy(g_hbm.at[srcbuf.at[0]], rows[0], gsems[0])
        for d in zds:
            d.wait()
        plsc.subcore_barrier()
        for j in range(1, cpt):
            if j >= 2:
                sd[j % 2].wait()
            gd[j % 2] = pltpu.async_copy(g_hbm.at[srcbuf.at[j]], rows[j % 2],
                                         gsems[j % 2])
            gd[(j - 1) % 2].wait()
            sd[(j - 1) % 2] = pltpu.async_copy(
                rows[(j - 1) % 2], acc.at[dstbuf.at[j - 1]],
                ssems[(j - 1) % 2], add=True)
        last = cpt - 1
        gd[last % 2].wait()
        sd[last % 2] = pltpu.async_copy(rows[last % 2], acc.at[dstbuf.at[last]],
                                        ssems[last % 2], add=True)
        sd[(last - 1) % 2].wait()
        sd[last % 2].wait()
        plsc.subcore_barrier()
        pltpu.sync_copy(
            acc.at[pl.ds(s * stripe, stripe)],
            out_hbm.at[pl.ds(c * n_pad + s * stripe, stripe)],
        )

    return scat_kernel


def _dinv_block(dega_ref, degb_ref):
    deg = dega_ref[:, 0:1] + degb_ref[:, 0:1] + 1.0
    return lax.rsqrt(jnp.maximum(deg, 1.0))


def _tc0_call(x, w1, n, n_pad):
    d_in, hid = w1.shape
    nb = n_pad // BLK

    def body(x_ref, w_ref, o_ref):
        i = pl.program_id(0)
        m = jnp.dot(x_ref[...], w_ref[...], preferred_element_type=jnp.float32)
        row = lax.broadcasted_iota(jnp.int32, (BLK, 1), 0) + i * BLK
        o_ref[...] = jnp.where(row < n, m, 0.0)

    return pl.pallas_call(
        body,
        grid=(nb,),
        in_specs=[
            pl.BlockSpec((BLK, d_in), lambda i: (i, 0)),
            pl.BlockSpec((d_in, hid), lambda i: (0, 0)),
        ],
        out_specs=pl.BlockSpec((BLK, hid), lambda i: (i, 0)),
        out_shape=jax.ShapeDtypeStruct((n_pad, hid), jnp.float32),
    )(x, w1)


def _scale_call(m1, degp, n_pad, hid):
    nb = n_pad // BLK

    def body(m_ref, dega_ref, degb_ref, g_ref, dinv_ref):
        dinv = _dinv_block(dega_ref, degb_ref)
        g_ref[...] = m_ref[...] * dinv
        dinv_ref[...] = dinv

    return pl.pallas_call(
        body,
        grid=(nb,),
        in_specs=[
            pl.BlockSpec((BLK, hid), lambda i: (i, 0)),
            pl.BlockSpec((BLK, hid), lambda i: (i, 0)),
            pl.BlockSpec((BLK, hid), lambda i: (i + nb, 0)),
        ],
        out_specs=[
            pl.BlockSpec((BLK, hid), lambda i: (i, 0)),
            pl.BlockSpec((BLK, 1), lambda i: (i, 0)),
        ],
        out_shape=[
            jax.ShapeDtypeStruct((n_pad, hid), jnp.float32),
            jax.ShapeDtypeStruct((n_pad, 1), jnp.float32),
        ],
    )(m1, degp, degp)


def _tc2_call(g1, parts, dinv_arr, b1, w2, n, n_pad):
    hid, hid2 = w2.shape
    nb = n_pad // BLK

    def body(g_ref, p0_ref, p1_ref, dinv_ref, b_ref, w_ref, o_ref):
        i = pl.program_id(0)
        dinv = dinv_ref[...]
        h = jnp.maximum(dinv * (g_ref[...] + p0_ref[...] + p1_ref[...])
                        + b_ref[...], 0.0)
        m = jnp.dot(h, w_ref[...], preferred_element_type=jnp.float32)
        row = lax.broadcasted_iota(jnp.int32, (BLK, 1), 0) + i * BLK
        o_ref[...] = jnp.where(row < n, m * dinv, 0.0)

    return pl.pallas_call(
        body,
        grid=(nb,),
        in_specs=[
            pl.BlockSpec((BLK, hid), lambda i: (i, 0)),
            pl.BlockSpec((BLK, hid), lambda i: (i, 0)),
            pl.BlockSpec((BLK, hid), lambda i: (i + nb, 0)),
            pl.BlockSpec((BLK, 1), lambda i: (i, 0)),
            pl.BlockSpec((1, hid), lambda i: (0, 0)),
            pl.BlockSpec((hid, hid2), lambda i: (0, 0)),
        ],
        out_specs=pl.BlockSpec((BLK, hid2), lambda i: (i, 0)),
        out_shape=jax.ShapeDtypeStruct((n_pad, hid2), jnp.float32),
    )(g1, parts, parts, dinv_arr, b1.reshape(1, hid), w2)


def _tc3_call(g2, parts, dinv_arr, b2, wc, bc, n, n_pad):
    hid, ncls = wc.shape
    nb = n_pad // BLK

    def body(g_ref, p0_ref, p1_ref, dinv_ref, b_ref, w_ref, bc_ref, o_ref):
        dinv = dinv_ref[...]
        h = jnp.maximum(dinv * (g_ref[...] + p0_ref[...] + p1_ref[...])
                        + b_ref[...], 0.0)
        o_ref[...] = jnp.dot(h, w_ref[...], preferred_element_type=jnp.float32) + bc_ref[...]

    return pl.pallas_call(
        body,
        grid=(nb,),
        in_specs=[
            pl.BlockSpec((BLK, hid), lambda i: (i, 0)),
            pl.BlockSpec((BLK, hid), lambda i: (i, 0)),
            pl.BlockSpec((BLK, hid), lambda i: (i + nb, 0)),
            pl.BlockSpec((BLK, 1), lambda i: (i, 0)),
            pl.BlockSpec((1, hid), lambda i: (0, 0)),
            pl.BlockSpec((hid, ncls), lambda i: (0, 0)),
            pl.BlockSpec((1, ncls), lambda i: (0, 0)),
        ],
        out_specs=pl.BlockSpec((BLK, ncls), lambda i: (i, 0)),
        out_shape=jax.ShapeDtypeStruct((n, ncls), jnp.float32),
    )(g2, parts, parts, dinv_arr, b2.reshape(1, hid), wc, bc.reshape(1, ncls))


def kernel(x, edge_index, W1, b1, W2, b2, Wc, bc):
    n, _ = x.shape
    hid = W1.shape[1]
    e = edge_index.shape[1]

    n_pad = _pad_to(n + 1, NS * ZR)

    ck = None
    for cand in range(128, 0, -1):
        if e % cand == 0 and (e // cand) % (NT * 8) == 0:
            ck = cand
            break
    if ck is not None:
        ei3 = edge_index.reshape(2, e // ck, ck)
    else:
        ck = 128
        e_pad = _pad_to(e, NT * ck * 8)
        pad_idx = n + jnp.arange(e_pad - e, dtype=jnp.int32) % (n_pad - n)
        pad2 = jnp.broadcast_to(pad_idx, (2, e_pad - e))
        ei3 = jnp.concatenate([edge_index, pad2], axis=1).reshape(
            2, e_pad // ck, ck)
    cpt = ei3.shape[1] // NT

    degp = _make_deg_kernel(n_pad, cpt, ck, hid)(ei3)
    m1 = _tc0_call(x, W1, n, n_pad)
    g1, dinv_arr = _scale_call(m1, degp, n_pad, hid)
    scat = _make_scat_kernel(n_pad, hid, cpt, ck)
    parts1 = scat(g1, ei3)
    g2 = _tc2_call(g1, parts1, dinv_arr, b1, W2, n, n_pad)
    parts2 = scat(g2, ei3)
    return _tc3_call(g2, parts2, dinv_arr, b2, Wc, bc, n, n_pad)
